# Initial kernel scaffold; baseline (speedup 1.0000x reference)
import jax
import jax.numpy as jnp
from jax.experimental import pallas as pl


def kernel(a_stream, omega_stream, params):
    raise NotImplementedError("write your pallas kernel here")


# speedup vs baseline: 77.2729x; 77.2729x over previous
import functools
import math

import jax
import jax.numpy as jnp
from jax.experimental import pallas as pl
from jax.experimental.pallas import tpu as pltpu

_NUM_HEADS = 8


def _ln(x, g, b):
    mu = jnp.mean(x, axis=-1, keepdims=True)
    var = jnp.mean((x - mu) ** 2, axis=-1, keepdims=True)
    return (x - mu) * jax.lax.rsqrt(var + 1e-5) * g + b


def _dot(a, b):
    return jnp.dot(a, b, preferred_element_type=jnp.float32)


def _shift_down(x):
    return jnp.concatenate([jnp.zeros_like(x[:1]), x[:-1]], axis=0)


def _shift_up(x):
    return jnp.concatenate([x[1:], jnp.zeros_like(x[:1])], axis=0)


def _attn_block(h, wq, wk, wv, wo, g, b, sel, selT):
    t, d = h.shape
    dh = d // _NUM_HEADS
    scale = 1.0 / math.sqrt(dh)
    q = _dot(h, wq)
    k = _dot(h, wk)
    v = _dot(h, wv)
    k_m1 = _shift_down(k)
    k_p1 = _shift_up(k)
    v_m1 = _shift_down(v)
    v_p1 = _shift_up(v)
    ls = _dot(q * k, sel) * scale
    ll = _dot(q * k_m1, sel) * scale
    lr = _dot(q * k_p1, sel) * scale
    row = jax.lax.broadcasted_iota(jnp.int32, ls.shape, 0)
    neg = jnp.float32(-1e30)
    ll = jnp.where(row == 0, neg, ll)
    lr = jnp.where(row == t - 1, neg, lr)
    m = jnp.maximum(ls, jnp.maximum(ll, lr))
    es = jnp.exp(ls - m)
    el = jnp.exp(ll - m)
    er = jnp.exp(lr - m)
    inv = 1.0 / (es + el + er + 1e-9)
    ws = _dot(es * inv, selT)
    wl = _dot(el * inv, selT)
    wr = _dot(er * inv, selT)
    out = ws * v + wl * v_m1 + wr * v_p1
    y = _dot(out, wo)
    return _ln(h + y, g, b)


def _branch(s, wrefs, sel, selT):
    in_w, in_b = wrefs[0][...], wrefs[1][...]
    h = _dot(s, in_w) + in_b
    idx = 2
    for _ in range(3):
        wq, wk, wv, wo, g, b = (r[...] for r in wrefs[idx:idx + 6])
        h = _attn_block(h, wq, wk, wv, wo, g, b, sel, selT)
        idx += 6
    return _ln(h, wrefs[idx][...], wrefs[idx + 1][...])


def _body(a_ref, o_ref, *refs):
    wrefs = refs[:-2]
    hg_ref, sensor_ref = refs[-2], refs[-1]

    d = wrefs[0].shape[1]
    dh = d // _NUM_HEADS
    ri = jax.lax.broadcasted_iota(jnp.int32, (d, _NUM_HEADS), 0)
    ci = jax.lax.broadcasted_iota(jnp.int32, (d, _NUM_HEADS), 1)
    sel = (ri // dh == ci).astype(jnp.float32)
    ri2 = jax.lax.broadcasted_iota(jnp.int32, (_NUM_HEADS, d), 0)
    ci2 = jax.lax.broadcasted_iota(jnp.int32, (_NUM_HEADS, d), 1)
    selT = (ci2 // dh == ri2).astype(jnp.float32)

    n_branch = 2 + 3 * 6 + 2
    a_tok = _branch(a_ref[0], wrefs[:n_branch], sel, selT)
    o_tok = _branch(o_ref[0], wrefs[n_branch:2 * n_branch], sel, selT)

    w1 = wrefs[2 * n_branch][...]
    b1 = wrefs[2 * n_branch + 1][...]
    w2 = wrefs[2 * n_branch + 2][...]
    b2 = wrefs[2 * n_branch + 3][...]
    x = _dot(a_tok, w1[:d, :]) + _dot(o_tok, w1[d:, :]) + b1
    x = jax.nn.gelu(x)
    sensor = _dot(x, w2) + b2
    sensor_ref[0] = sensor
    hg_ref[0] = jnp.mean(sensor, axis=0, keepdims=True)


def _flatten_params(params):
    arrs = []
    for name in ('a', 'omega'):
        p = params[name]
        arrs += [p['in_w'], p['in_b'].reshape(1, -1)]
        for blk in p['blocks']:
            arrs += [blk['wq'], blk['wk'], blk['wv'], blk['wo'],
                     blk['ln_g'].reshape(1, -1), blk['ln_b'].reshape(1, -1)]
        arrs += [p['norm_g'].reshape(1, -1), p['norm_b'].reshape(1, -1)]
    arrs += [params['fuse_w1'], params['fuse_b1'].reshape(1, -1),
             params['fuse_w2'], params['fuse_b2'].reshape(1, -1)]
    return arrs


@jax.jit
def kernel(a_stream, omega_stream, params):
    bsz, t, cin = a_stream.shape
    d = params['fuse_w2'].shape[0]
    warrs = _flatten_params(params)

    def const_spec(w):
        nd = w.ndim
        return pl.BlockSpec(w.shape, lambda b, _n=nd: (0,) * _n)

    hg, sensor = pl.pallas_call(
        _body,
        grid=(bsz,),
        in_specs=[
            pl.BlockSpec((1, t, cin), lambda b: (b, 0, 0)),
            pl.BlockSpec((1, t, cin), lambda b: (b, 0, 0)),
        ] + [const_spec(w) for w in warrs],
        out_specs=[
            pl.BlockSpec((1, 1, d), lambda b: (b, 0, 0)),
            pl.BlockSpec((1, t, d), lambda b: (b, 0, 0)),
        ],
        out_shape=[
            jax.ShapeDtypeStruct((bsz, 1, d), jnp.float32),
            jax.ShapeDtypeStruct((bsz, t, d), jnp.float32),
        ],
        compiler_params=pltpu.CompilerParams(
            dimension_semantics=("arbitrary",),
            vmem_limit_bytes=120 * 1024 * 1024,
        ),
    )(a_stream, omega_stream, *warrs)
    return hg.reshape(bsz, d), sensor
